# baseline (speedup 1.0000x reference)
import functools

import jax
import jax.numpy as jnp
from jax.experimental import pallas as pl
from jax.experimental.pallas import tpu as pltpu

_BN = 1000
_NBUF = 4


def _gcn_body(x_ref, s_ref, w_ref, b_ref, o_hbm, obuf, c_ref, sems):
    k = pl.program_id(0)
    nsteps = pl.num_programs(0)

    @pl.when(k == 0)
    def _():
        c32 = jax.lax.dot_general(
            s_ref[:], w_ref[:], (((1,), (1,)), ((), ())),
            preferred_element_type=jnp.float32)
        c_ref[:] = c32.astype(jnp.bfloat16)

    x_bf = x_ref[:].astype(jnp.bfloat16)
    acc = jnp.dot(x_bf, c_ref[:], preferred_element_type=jnp.float32)
    res = jnp.maximum(acc + b_ref[:], 0.0)

    slot = jax.lax.rem(k, _NBUF)

    @pl.when(k >= _NBUF)
    def _():
        pltpu.make_async_copy(
            obuf.at[slot],
            o_hbm.at[pl.ds((k - _NBUF) * _BN, _BN), :],
            sems.at[slot]).wait()

    obuf[slot] = res
    pltpu.make_async_copy(
        obuf.at[slot],
        o_hbm.at[pl.ds(k * _BN, _BN), :],
        sems.at[slot]).start()

    @pl.when(k == nsteps - 1)
    def _():
        for j in range(_NBUF):
            kk = k - (_NBUF - 1) + j
            pltpu.make_async_copy(
                obuf.at[jax.lax.rem(kk, _NBUF)],
                o_hbm.at[pl.ds(kk * _BN, _BN), :],
                sems.at[jax.lax.rem(kk, _NBUF)]).wait()


@functools.partial(jax.jit, static_argnames=())
def kernel(x, support, W, b):
    n, d = x.shape
    out_c, in_c = W.shape
    bn = _BN
    out = pl.pallas_call(
        _gcn_body,
        grid=(n // bn,),
        in_specs=[
            pl.BlockSpec((bn, d), lambda i: (i, 0)),
            pl.BlockSpec((d, in_c), lambda i: (0, 0)),
            pl.BlockSpec((out_c, in_c), lambda i: (0, 0)),
            pl.BlockSpec((1, out_c), lambda i: (0, 0)),
        ],
        out_specs=pl.BlockSpec(memory_space=pltpu.MemorySpace.HBM),
        out_shape=jax.ShapeDtypeStruct((n, out_c), jnp.float32),
        scratch_shapes=[
            pltpu.VMEM((_NBUF, bn, out_c), jnp.float32),
            pltpu.VMEM((d, out_c), jnp.bfloat16),
            pltpu.SemaphoreType.DMA((_NBUF,)),
        ],
        compiler_params=pltpu.CompilerParams(
            vmem_limit_bytes=120 * 1024 * 1024),
    )(x, support, W, b.reshape(1, out_c))
    return out


# speedup vs baseline: 1.0566x; 1.0566x over previous
import functools

import jax
import jax.numpy as jnp
from jax.experimental import pallas as pl
from jax.experimental.pallas import tpu as pltpu

_BN = 2000
_NBUF = 3


def _gcn_body(x_ref, s_ref, w_ref, b_ref, o_hbm, obuf, c_ref, sems):
    k = pl.program_id(0)
    nsteps = pl.num_programs(0)

    @pl.when(k == 0)
    def _():
        c32 = jax.lax.dot_general(
            s_ref[:], w_ref[:], (((1,), (1,)), ((), ())),
            preferred_element_type=jnp.float32)
        c_ref[:] = c32.astype(jnp.bfloat16)

    x_bf = x_ref[:].astype(jnp.bfloat16)
    acc = jnp.dot(x_bf, c_ref[:], preferred_element_type=jnp.float32)
    res = jnp.maximum(acc + b_ref[:], 0.0)

    slot = jax.lax.rem(k, _NBUF)

    @pl.when(k >= _NBUF)
    def _():
        pltpu.make_async_copy(
            obuf.at[slot],
            o_hbm.at[pl.ds((k - _NBUF) * _BN, _BN), :],
            sems.at[slot]).wait()

    obuf[slot] = res
    pltpu.make_async_copy(
        obuf.at[slot],
        o_hbm.at[pl.ds(k * _BN, _BN), :],
        sems.at[slot]).start()

    @pl.when(k == nsteps - 1)
    def _():
        for j in range(_NBUF):
            kk = k - (_NBUF - 1) + j
            pltpu.make_async_copy(
                obuf.at[jax.lax.rem(kk, _NBUF)],
                o_hbm.at[pl.ds(kk * _BN, _BN), :],
                sems.at[jax.lax.rem(kk, _NBUF)]).wait()


@functools.partial(jax.jit, static_argnames=())
def kernel(x, support, W, b):
    n, d = x.shape
    out_c, in_c = W.shape
    bn = _BN
    out = pl.pallas_call(
        _gcn_body,
        grid=(n // bn,),
        in_specs=[
            pl.BlockSpec((bn, d), lambda i: (i, 0)),
            pl.BlockSpec((d, in_c), lambda i: (0, 0)),
            pl.BlockSpec((out_c, in_c), lambda i: (0, 0)),
            pl.BlockSpec((1, out_c), lambda i: (0, 0)),
        ],
        out_specs=pl.BlockSpec(memory_space=pltpu.MemorySpace.HBM),
        out_shape=jax.ShapeDtypeStruct((n, out_c), jnp.float32),
        scratch_shapes=[
            pltpu.VMEM((_NBUF, bn, out_c), jnp.float32),
            pltpu.VMEM((d, out_c), jnp.bfloat16),
            pltpu.SemaphoreType.DMA((_NBUF,)),
        ],
        compiler_params=pltpu.CompilerParams(
            vmem_limit_bytes=120 * 1024 * 1024),
    )(x, support, W, b.reshape(1, out_c))
    return out


# speedup vs baseline: 1.1233x; 1.0632x over previous
import functools

import jax
import jax.numpy as jnp
from jax.experimental import pallas as pl
from jax.experimental.pallas import tpu as pltpu

_BN = 2000


def _gcn_body(xl_ref, xr_ref, s_ref, w_ref, b_ref, o_ref, c_ref):
    i = pl.program_id(0)

    @pl.when(i == 0)
    def _():
        c32 = jax.lax.dot_general(
            s_ref[:], w_ref[:], (((1,), (1,)), ((), ())),
            preferred_element_type=jnp.float32)
        c_ref[:] = c32.astype(jnp.bfloat16)

    d2 = xl_ref.shape[1]
    accl = jnp.dot(xl_ref[:].astype(jnp.bfloat16), c_ref[:d2],
                   preferred_element_type=jnp.float32)
    accr = jnp.dot(xr_ref[:].astype(jnp.bfloat16), c_ref[d2:],
                   preferred_element_type=jnp.float32)
    o_ref[:] = jnp.maximum(accl + accr + b_ref[:], 0.0)


@functools.partial(jax.jit, static_argnames=())
def kernel(x, support, W, b):
    n, d = x.shape
    out_c, in_c = W.shape
    bn = _BN
    d2 = d // 2
    out = pl.pallas_call(
        _gcn_body,
        grid=(n // bn,),
        in_specs=[
            pl.BlockSpec((bn, d2), lambda i: (i, 0)),
            pl.BlockSpec((bn, d2), lambda i: (i, 1)),
            pl.BlockSpec((d, in_c), lambda i: (0, 0)),
            pl.BlockSpec((out_c, in_c), lambda i: (0, 0)),
            pl.BlockSpec((1, out_c), lambda i: (0, 0)),
        ],
        out_specs=pl.BlockSpec((bn, out_c), lambda i: (i, 0)),
        out_shape=jax.ShapeDtypeStruct((n, out_c), jnp.float32),
        scratch_shapes=[pltpu.VMEM((d, out_c), jnp.bfloat16)],
        compiler_params=pltpu.CompilerParams(
            vmem_limit_bytes=120 * 1024 * 1024),
    )(x, x, support, W, b.reshape(1, out_c))
    return out


# speedup vs baseline: 1.2858x; 1.1447x over previous
import functools

import jax
import jax.numpy as jnp
from jax.experimental import pallas as pl
from jax.experimental.pallas import tpu as pltpu

_BN = 1000
_NBUF = 4


def _gcn_body(x_hbm, s_ref, w_ref, b_ref, o_hbm,
              xbuf, obuf, c_ref, insems, outsems):
    nblk = x_hbm.shape[0] // _BN

    def in_copy(k):
        return pltpu.make_async_copy(
            x_hbm.at[pl.ds(k * _BN, _BN), :],
            xbuf.at[k % _NBUF],
            insems.at[k % _NBUF])

    def out_copy(k):
        return pltpu.make_async_copy(
            obuf.at[k % _NBUF],
            o_hbm.at[pl.ds(k * _BN, _BN), :],
            outsems.at[k % _NBUF])

    for k in range(_NBUF):
        in_copy(k).start()

    c32 = jax.lax.dot_general(
        s_ref[:], w_ref[:], (((1,), (1,)), ((), ())),
        preferred_element_type=jnp.float32)
    c_ref[:] = c32.astype(jnp.bfloat16)

    for k in range(nblk):
        slot = k % _NBUF
        in_copy(k).wait()
        acc = jnp.dot(xbuf[slot].astype(jnp.bfloat16), c_ref[:],
                      preferred_element_type=jnp.float32)
        res = jnp.maximum(acc + b_ref[:], 0.0)
        if k >= _NBUF:
            out_copy(k - _NBUF).wait()
        obuf[slot] = res
        out_copy(k).start()
        if k + _NBUF < nblk:
            in_copy(k + _NBUF).start()

    for k in range(nblk - _NBUF, nblk):
        out_copy(k).wait()


@functools.partial(jax.jit, static_argnames=())
def kernel(x, support, W, b):
    n, d = x.shape
    out_c, in_c = W.shape
    out = pl.pallas_call(
        _gcn_body,
        in_specs=[
            pl.BlockSpec(memory_space=pltpu.MemorySpace.HBM),
            pl.BlockSpec(memory_space=pltpu.MemorySpace.VMEM),
            pl.BlockSpec(memory_space=pltpu.MemorySpace.VMEM),
            pl.BlockSpec(memory_space=pltpu.MemorySpace.VMEM),
        ],
        out_specs=pl.BlockSpec(memory_space=pltpu.MemorySpace.HBM),
        out_shape=jax.ShapeDtypeStruct((n, out_c), jnp.float32),
        scratch_shapes=[
            pltpu.VMEM((_NBUF, _BN, d), jnp.float32),
            pltpu.VMEM((_NBUF, _BN, out_c), jnp.float32),
            pltpu.VMEM((d, out_c), jnp.bfloat16),
            pltpu.SemaphoreType.DMA((_NBUF,)),
            pltpu.SemaphoreType.DMA((_NBUF,)),
        ],
        compiler_params=pltpu.CompilerParams(
            vmem_limit_bytes=120 * 1024 * 1024),
    )(x, support, W, b.reshape(1, out_c))
    return out


# speedup vs baseline: 1.3516x; 1.0511x over previous
import functools

import jax
import jax.numpy as jnp
from jax.experimental import pallas as pl
from jax.experimental.pallas import tpu as pltpu

_BN = 1000
_NBUF = 6


def _gcn_body(x_hbm, s_ref, w_ref, b_ref, o_hbm,
              xbuf, obuf, c_ref, insems, outsems):
    nblk = x_hbm.shape[0] // _BN

    def in_copy(k):
        return pltpu.make_async_copy(
            x_hbm.at[pl.ds(k * _BN, _BN), :],
            xbuf.at[k % _NBUF],
            insems.at[k % _NBUF])

    def out_copy(k):
        return pltpu.make_async_copy(
            obuf.at[k % _NBUF],
            o_hbm.at[pl.ds(k * _BN, _BN), :],
            outsems.at[k % _NBUF])

    for k in range(_NBUF):
        in_copy(k).start()

    c32 = jax.lax.dot_general(
        s_ref[:], w_ref[:], (((1,), (1,)), ((), ())),
        preferred_element_type=jnp.float32)
    c_ref[:] = c32.astype(jnp.bfloat16)

    for k in range(nblk):
        slot = k % _NBUF
        in_copy(k).wait()
        acc = jnp.dot(xbuf[slot].astype(jnp.bfloat16), c_ref[:],
                      preferred_element_type=jnp.float32)
        res = jnp.maximum(acc + b_ref[:], 0.0)
        if k >= _NBUF:
            out_copy(k - _NBUF).wait()
        obuf[slot] = res
        out_copy(k).start()
        if k + _NBUF < nblk:
            in_copy(k + _NBUF).start()

    for k in range(nblk - _NBUF, nblk):
        out_copy(k).wait()


@functools.partial(jax.jit, static_argnames=())
def kernel(x, support, W, b):
    n, d = x.shape
    out_c, in_c = W.shape
    out = pl.pallas_call(
        _gcn_body,
        in_specs=[
            pl.BlockSpec(memory_space=pltpu.MemorySpace.HBM),
            pl.BlockSpec(memory_space=pltpu.MemorySpace.VMEM),
            pl.BlockSpec(memory_space=pltpu.MemorySpace.VMEM),
            pl.BlockSpec(memory_space=pltpu.MemorySpace.VMEM),
        ],
        out_specs=pl.BlockSpec(memory_space=pltpu.MemorySpace.HBM),
        out_shape=jax.ShapeDtypeStruct((n, out_c), jnp.float32),
        scratch_shapes=[
            pltpu.VMEM((_NBUF, _BN, d), jnp.float32),
            pltpu.VMEM((_NBUF, _BN, out_c), jnp.float32),
            pltpu.VMEM((d, out_c), jnp.bfloat16),
            pltpu.SemaphoreType.DMA((_NBUF,)),
            pltpu.SemaphoreType.DMA((_NBUF,)),
        ],
        compiler_params=pltpu.CompilerParams(
            vmem_limit_bytes=120 * 1024 * 1024),
    )(x, support, W, b.reshape(1, out_c))
    return out
